# baseline (speedup 1.0000x reference)
import jax
import jax.numpy as jnp
from jax.experimental import pallas as pl
from jax.experimental.pallas import tpu as pltpu

_PALETTE_KEYS = (
    (0 << 16) | (0 << 8) | 0,
    (128 << 16) | (0 << 8) | 0,
    (128 << 16) | (64 << 8) | 128,
    (192 << 16) | (0 << 8) | 192,
    (0 << 16) | (128 << 8) | 0,
    (128 << 16) | (128 << 8) | 0,
    (64 << 16) | (64 << 8) | 0,
    (64 << 16) | (0 << 8) | 128,
)

_BLOCK_ROWS = 80


def _lookup_kernel(t_ref, o_ref):
    r = t_ref[0]
    g = t_ref[1]
    b = t_ref[2]
    keys = (r << 16) | (g << 8) | b
    out = jnp.zeros(keys.shape, jnp.int32)
    for cid in range(1, 8):
        out = jnp.where(keys == _PALETTE_KEYS[cid], cid, out)
    o_ref[...] = out.astype(jnp.uint8)


def kernel(tensor):
    c, h, w = tensor.shape
    block_rows = _BLOCK_ROWS if h % _BLOCK_ROWS == 0 else h
    grid = h // block_rows
    return pl.pallas_call(
        _lookup_kernel,
        grid=(grid,),
        in_specs=[pl.BlockSpec((3, block_rows, w), lambda i: (0, i, 0))],
        out_specs=pl.BlockSpec((block_rows, w), lambda i: (i, 0)),
        out_shape=jax.ShapeDtypeStruct((h, w), jnp.uint8),
        compiler_params=pltpu.CompilerParams(
            dimension_semantics=("parallel",),
        ),
    )(tensor)


# speedup vs baseline: 1.2077x; 1.2077x over previous
import jax
import jax.numpy as jnp
from jax.experimental import pallas as pl
from jax.experimental.pallas import tpu as pltpu

_PALETTE_KEYS = (
    (0 << 16) | (0 << 8) | 0,
    (128 << 16) | (0 << 8) | 0,
    (128 << 16) | (64 << 8) | 128,
    (192 << 16) | (0 << 8) | 192,
    (0 << 16) | (128 << 8) | 0,
    (128 << 16) | (128 << 8) | 0,
    (64 << 16) | (64 << 8) | 0,
    (64 << 16) | (0 << 8) | 128,
)

_BLOCK_ROWS = 216


def _lookup_kernel(t_ref, o_ref):
    r = t_ref[0]
    g = t_ref[1]
    b = t_ref[2]
    keys = (r << 16) | (g << 8) | b
    out = jnp.zeros(keys.shape, jnp.int32)
    for cid in range(1, 8):
        out = jnp.where(keys == _PALETTE_KEYS[cid], cid, out)
    o_ref[...] = out.astype(jnp.uint8)


def kernel(tensor):
    c, h, w = tensor.shape
    block_rows = _BLOCK_ROWS if h % _BLOCK_ROWS == 0 else h
    grid = h // block_rows
    return pl.pallas_call(
        _lookup_kernel,
        grid=(grid,),
        in_specs=[pl.BlockSpec((3, block_rows, w), lambda i: (0, i, 0))],
        out_specs=pl.BlockSpec((block_rows, w), lambda i: (i, 0)),
        out_shape=jax.ShapeDtypeStruct((h, w), jnp.uint8),
        compiler_params=pltpu.CompilerParams(
            dimension_semantics=("parallel",),
        ),
    )(tensor)
